# baseline (speedup 1.0000x reference)
import functools

import jax
import jax.numpy as jnp
from jax import lax
from jax.experimental import pallas as pl
from jax.experimental.pallas import tpu as pltpu
from jax.experimental.pallas import tpu_sc as plsc

_H, _W = 512, 512
_B, _V, _F, _VT = 4, 10000, 20000, 12000
_NW = 32
_NPIX = _B * _H * _W
_CHUNK = 4096

_mesh = plsc.VectorSubcoreMesh(core_axis_name="c", subcore_axis_name="s")
_params = pltpu.CompilerParams(
    needs_layout_passes=False, use_tc_tiling_on_sc=False
)


def _eclamp(x):
    return jnp.where(x < 0, jnp.minimum(x, -1e-8), jnp.maximum(x, 1e-8))


def _worker():
    wid = lax.axis_index("s") * 2 + lax.axis_index("c")
    b = wid // 8
    y0 = (wid % 8) * 64
    rb0 = b * 64 + (y0 >> 3)
    return b, y0, rb0


def _plane_off(b, rb, c):
    return ((b * 2 + c) * 64 + rb) * _CHUNK


@functools.partial(
    pl.kernel,
    mesh=_mesh,
    compiler_params=_params,
    out_type=(
        jax.ShapeDtypeStruct((_NPIX,), jnp.float32),
        jax.ShapeDtypeStruct((_NPIX * 3,), jnp.float32),
    ),
    scratch_types=[
        pltpu.VMEM((_V * 3,), jnp.float32),
        pltpu.VMEM((_F * 3,), jnp.int32),
        pltpu.VMEM((_CHUNK,), jnp.int32),
        pltpu.VMEM((_CHUNK,), jnp.int32),
        pltpu.VMEM((_CHUNK,), jnp.float32),
        pltpu.VMEM((_CHUNK,), jnp.float32),
        pltpu.VMEM((_CHUNK,), jnp.float32),
        pltpu.VMEM((_CHUNK,), jnp.float32),
        pltpu.SemaphoreType.DMA,
        pltpu.SemaphoreType.DMA,
        pltpu.SemaphoreType.DMA,
    ],
)
def _geom(v2d_hbm, vi_hbm, idx_hbm, depth_hbm, bary_hbm,
          v2d_v, vi_v, idx_a, idx_b, dep_v, q0_v, q1_v, q2_v,
          semt, semi, semo):
    b, y0, rb0 = _worker()
    loads = [
        pltpu.async_copy(v2d_hbm.at[b], v2d_v, semt),
        pltpu.async_copy(vi_hbm, vi_v, semt),
    ]
    for h in loads:
        h.wait()
    iota = lax.iota(jnp.int32, 16)

    def outs(rb):
        return [
            (dep_v, depth_hbm, rb * _CHUNK),
            (q0_v, bary_hbm, _plane_off(b, rb, 0)),
            (q1_v, bary_hbm, _plane_off(b, rb, 1)),
            (q2_v, bary_hbm, _plane_off(b, rb, 2)),
        ]

    idx_bufs = (idx_a, idx_b)
    pltpu.async_copy(idx_hbm.at[pl.ds(rb0 * _CHUNK, _CHUNK)], idx_a, semi)

    def chunk_pair(t, carry):
      for par in range(2):
        ci = t * 2 + par
        yc = y0 + ci * 8
        rb = rb0 + ci
        idx_v = idx_bufs[par]
        nxt = idx_bufs[1 - par]
        pltpu.make_async_copy(
            idx_hbm.at[pl.ds(rb * _CHUNK, _CHUNK)], idx_v, semi
        ).wait()

        @pl.when(ci < 7)
        def _():
            pltpu.async_copy(
                idx_hbm.at[pl.ds((rb + 1) * _CHUNK, _CHUNK)], nxt, semi
            )

        @pl.when(ci > 0)
        def _():
            for src, dst, off in outs(rb):
                pltpu.make_async_copy(src, dst.at[pl.ds(off, _CHUNK)], semo).wait()

        def vec_body(mi, carry2):
            for half in range(4):
                l = mi * 64 + half * 16
                f3 = idx_v[pl.ds(l, 16)] * 3
                i0 = plsc.load_gather(vi_v, [f3]) * 3
                i1 = plsc.load_gather(vi_v, [f3 + 1]) * 3
                i2 = plsc.load_gather(vi_v, [f3 + 2]) * 3
                v0x = plsc.load_gather(v2d_v, [i0])
                v0y = plsc.load_gather(v2d_v, [i0 + 1])
                v0z = plsc.load_gather(v2d_v, [i0 + 2])
                v1x = plsc.load_gather(v2d_v, [i1])
                v1y = plsc.load_gather(v2d_v, [i1 + 1])
                v1z = plsc.load_gather(v2d_v, [i1 + 2])
                v2x = plsc.load_gather(v2d_v, [i2])
                v2y = plsc.load_gather(v2d_v, [i2 + 1])
                v2z = plsc.load_gather(v2d_v, [i2 + 2])
                xb = ((l >> 10) << 7) | (l & 127)
                x = (iota + xb).astype(jnp.float32)
                y = (yc + ((l >> 7) & 7)).astype(jnp.float32)
                e1x = v1x - v0x
                e1y = v1y - v0y
                e2x = v2x - v0x
                e2y = v2y - v0y
                den = _eclamp(e1x * e2y - e1y * e2x)
                w0 = 1.0 / _eclamp(v0z)
                w1 = 1.0 / _eclamp(v1z)
                w2 = 1.0 / _eclamp(v2z)
                px = x - v0x
                py = y - v0y
                l1 = (px * e2y - py * e2x) / den
                l2 = (py * e1x - px * e1y) / den
                lam0 = 1.0 - l1 - l2
                u0 = w0 * lam0
                u1 = w1 * l1
                u2 = w2 * l2
                zi = 1.0 / _eclamp(u0 + u1 + u2)
                dep_v[pl.ds(l, 16)] = zi
                q0_v[pl.ds(l, 16)] = u0 * zi
                q1_v[pl.ds(l, 16)] = u1 * zi
                q2_v[pl.ds(l, 16)] = u2 * zi
            return carry2

        lax.fori_loop(0, _CHUNK // 64, vec_body, 0)
        for src, dst, off in outs(rb):
            pltpu.async_copy(src, dst.at[pl.ds(off, _CHUNK)], semo)
      return carry

    lax.fori_loop(0, 4, chunk_pair, 0)
    for src, dst, off in outs(rb0 + 7):
        pltpu.make_async_copy(src, dst.at[pl.ds(off, _CHUNK)], semo).wait()


@functools.partial(
    pl.kernel,
    mesh=_mesh,
    compiler_params=_params,
    out_type=jax.ShapeDtypeStruct((_NPIX * 2,), jnp.float32),
    scratch_types=[
        pltpu.VMEM((_VT * 2,), jnp.float32),
        pltpu.VMEM((_F * 3,), jnp.int32),
        pltpu.VMEM((_CHUNK,), jnp.int32),
        pltpu.VMEM((_CHUNK,), jnp.int32),
        pltpu.VMEM((_CHUNK,), jnp.float32),
        pltpu.VMEM((_CHUNK,), jnp.float32),
        pltpu.VMEM((_CHUNK,), jnp.float32),
        pltpu.VMEM((_CHUNK,), jnp.float32),
        pltpu.VMEM((_CHUNK,), jnp.float32),
        pltpu.VMEM((_CHUNK,), jnp.float32),
        pltpu.VMEM((_CHUNK * 2,), jnp.float32),
        pltpu.SemaphoreType.DMA,
        pltpu.SemaphoreType.DMA,
        pltpu.SemaphoreType.DMA,
    ],
)
def _texco(vt_hbm, vti_hbm, idx_hbm, bary_hbm, vto_hbm,
           vt_v, vti_v, idx_a, idx_b, qa0, qa1, qa2, qb0, qb1, qb2,
           out_v, semt, semi, semo):
    b, y0, rb0 = _worker()
    loads = [
        pltpu.async_copy(vt_hbm, vt_v, semt),
        pltpu.async_copy(vti_hbm, vti_v, semt),
    ]
    for h in loads:
        h.wait()

    insets = ((idx_a, qa0, qa1, qa2), (idx_b, qb0, qb1, qb2))

    def fire_ins(rb, bufs):
        pltpu.async_copy(idx_hbm.at[pl.ds(rb * _CHUNK, _CHUNK)], bufs[0], semi)
        for c in range(3):
            pltpu.async_copy(
                bary_hbm.at[pl.ds(_plane_off(b, rb, c), _CHUNK)],
                bufs[1 + c], semi)

    def wait_ins(rb, bufs):
        pltpu.make_async_copy(
            idx_hbm.at[pl.ds(rb * _CHUNK, _CHUNK)], bufs[0], semi).wait()
        for c in range(3):
            pltpu.make_async_copy(
                bary_hbm.at[pl.ds(_plane_off(b, rb, c), _CHUNK)],
                bufs[1 + c], semi).wait()

    fire_ins(rb0, insets[0])

    def chunk_pair(t, carry):
      for par in range(2):
        ci = t * 2 + par
        rb = rb0 + ci
        idx_v, q0_v, q1_v, q2_v = insets[par]
        wait_ins(rb, insets[par])

        @pl.when(ci < 7)
        def _():
            fire_ins(rb + 1, insets[1 - par])

        @pl.when(ci > 0)
        def _():
            pltpu.make_async_copy(
                out_v, vto_hbm.at[pl.ds(rb * _CHUNK * 2, _CHUNK * 2)], semo
            ).wait()

        def vec_body(mi, carry2):
            for half in range(4):
                l = mi * 64 + half * 16
                f3 = idx_v[pl.ds(l, 16)] * 3
                t0 = plsc.load_gather(vti_v, [f3]) * 2
                t1 = plsc.load_gather(vti_v, [f3 + 1]) * 2
                t2 = plsc.load_gather(vti_v, [f3 + 2]) * 2
                q0 = q0_v[pl.ds(l, 16)]
                q1 = q1_v[pl.ds(l, 16)]
                q2 = q2_v[pl.ds(l, 16)]
                vtx = (plsc.load_gather(vt_v, [t0]) * q0
                       + plsc.load_gather(vt_v, [t1]) * q1
                       + plsc.load_gather(vt_v, [t2]) * q2) * 2.0 - 1.0
                vty = (plsc.load_gather(vt_v, [t0 + 1]) * q0
                       + plsc.load_gather(vt_v, [t1 + 1]) * q1
                       + plsc.load_gather(vt_v, [t2 + 1]) * q2) * 2.0 - 1.0
                vtoff = ((l >> 7) & 7) * 1024 + (l >> 10) * 256 + (l & 127)
                out_v[pl.ds(vtoff, 16)] = vtx
                out_v[pl.ds(vtoff + 128, 16)] = vty
            return carry2

        lax.fori_loop(0, _CHUNK // 64, vec_body, 0)
        pltpu.async_copy(
            out_v, vto_hbm.at[pl.ds(rb * _CHUNK * 2, _CHUNK * 2)], semo
        )
      return carry

    lax.fori_loop(0, 4, chunk_pair, 0)
    pltpu.make_async_copy(
        out_v, vto_hbm.at[pl.ds((rb0 + 7) * _CHUNK * 2, _CHUNK * 2)], semo
    ).wait()


@functools.partial(
    pl.kernel,
    mesh=_mesh,
    compiler_params=_params,
    out_type=jax.ShapeDtypeStruct((_NPIX * 3,), jnp.float32),
    scratch_types=[
        pltpu.VMEM((_V * 3,), jnp.float32),
        pltpu.VMEM((_F * 3,), jnp.int32),
        pltpu.VMEM((_CHUNK,), jnp.int32),
        pltpu.VMEM((_CHUNK,), jnp.float32),
        pltpu.VMEM((_CHUNK,), jnp.float32),
        pltpu.VMEM((_CHUNK,), jnp.float32),
        pltpu.VMEM((_CHUNK,), jnp.float32),
        pltpu.VMEM((_CHUNK,), jnp.float32),
        pltpu.VMEM((_CHUNK,), jnp.float32),
        pltpu.SemaphoreType.DMA,
        pltpu.SemaphoreType.DMA,
        pltpu.SemaphoreType.DMA,
    ],
)
def _normals(vn_hbm, vi_hbm, idx_hbm, bary_hbm, vno_hbm,
             vn_v, vi_v, idx_v, q0_v, q1_v, q2_v, n0_v, n1_v, n2_v,
             semt, semi, semo):
    b, y0, rb0 = _worker()
    loads = [
        pltpu.async_copy(vn_hbm.at[b], vn_v, semt),
        pltpu.async_copy(vi_hbm, vi_v, semt),
    ]
    for h in loads:
        h.wait()

    def outs(rb):
        return [
            (n0_v, _plane_off(b, rb, 0)),
            (n1_v, _plane_off(b, rb, 1)),
            (n2_v, _plane_off(b, rb, 2)),
        ]

    def chunk_body(ci, carry):
        rb = rb0 + ci
        ins = [
            pltpu.async_copy(idx_hbm.at[pl.ds(rb * _CHUNK, _CHUNK)], idx_v, semi),
            pltpu.async_copy(
                bary_hbm.at[pl.ds(_plane_off(b, rb, 0), _CHUNK)], q0_v, semi),
            pltpu.async_copy(
                bary_hbm.at[pl.ds(_plane_off(b, rb, 1), _CHUNK)], q1_v, semi),
            pltpu.async_copy(
                bary_hbm.at[pl.ds(_plane_off(b, rb, 2), _CHUNK)], q2_v, semi),
        ]
        for h in ins:
            h.wait()

        @pl.when(ci > 0)
        def _():
            for src, off in outs(rb):
                pltpu.make_async_copy(
                    src, vno_hbm.at[pl.ds(off, _CHUNK)], semo
                ).wait()

        def vec_body(mi, carry2):
            for half in range(4):
                l = mi * 64 + half * 16
                f3 = idx_v[pl.ds(l, 16)] * 3
                i0 = plsc.load_gather(vi_v, [f3]) * 3
                i1 = plsc.load_gather(vi_v, [f3 + 1]) * 3
                i2 = plsc.load_gather(vi_v, [f3 + 2]) * 3
                q0 = q0_v[pl.ds(l, 16)]
                q1 = q1_v[pl.ds(l, 16)]
                q2 = q2_v[pl.ds(l, 16)]
                n0_v[pl.ds(l, 16)] = (
                    plsc.load_gather(vn_v, [i0]) * q0
                    + plsc.load_gather(vn_v, [i1]) * q1
                    + plsc.load_gather(vn_v, [i2]) * q2)
                n1_v[pl.ds(l, 16)] = (
                    plsc.load_gather(vn_v, [i0 + 1]) * q0
                    + plsc.load_gather(vn_v, [i1 + 1]) * q1
                    + plsc.load_gather(vn_v, [i2 + 1]) * q2)
                n2_v[pl.ds(l, 16)] = (
                    plsc.load_gather(vn_v, [i0 + 2]) * q0
                    + plsc.load_gather(vn_v, [i1 + 2]) * q1
                    + plsc.load_gather(vn_v, [i2 + 2]) * q2)
            return carry2

        lax.fori_loop(0, _CHUNK // 64, vec_body, 0)
        for src, off in outs(rb):
            pltpu.async_copy(src, vno_hbm.at[pl.ds(off, _CHUNK)], semo)
        return carry

    lax.fori_loop(0, 8, chunk_body, 0)
    for src, off in outs(rb0 + 7):
        pltpu.make_async_copy(src, vno_hbm.at[pl.ds(off, _CHUNK)], semo).wait()


def kernel(v2d, vt, vn, vi, vti, index_img):
    vi_f = vi.reshape(-1)
    vti_f = vti.reshape(-1)
    v2d_f = v2d.reshape(_B, _V * 3)
    vn_f = vn.reshape(_B, _V * 3)
    vt_f = vt.reshape(-1)
    idx_tiled = index_img.reshape(_B, 64, 8, 4, 128)
    idx_tiled = idx_tiled.transpose(0, 1, 3, 2, 4).reshape(-1)
    depth, bary = _geom(v2d_f, vi_f, idx_tiled)
    vto = _texco(vt_f, vti_f, idx_tiled, bary)
    vno = _normals(vn_f, vi_f, idx_tiled, bary)
    depth = depth.reshape(_B, 64, 4, 8, 128).transpose(0, 1, 3, 2, 4)
    depth = depth.reshape(_B, _H, _W)
    bary = bary.reshape(_B, 3, 64, 4, 8, 128).transpose(0, 2, 4, 3, 5, 1)
    bary = bary.reshape(_B, _H, _W, 3)
    vno = vno.reshape(_B, 3, 64, 4, 8, 128).transpose(0, 2, 4, 3, 5, 1)
    vno = vno.reshape(_B, _H, _W, 3)
    vto = vto.reshape(_B, _H, 4, 2, 128).transpose(0, 1, 2, 4, 3)
    vto = vto.reshape(_B, _H, _W, 2)
    return depth, bary, vto, vno


# speedup vs baseline: 1.0331x; 1.0331x over previous
import functools

import jax
import jax.numpy as jnp
from jax import lax
from jax.experimental import pallas as pl
from jax.experimental.pallas import tpu as pltpu
from jax.experimental.pallas import tpu_sc as plsc

_H, _W = 512, 512
_B, _V, _F, _VT = 4, 10000, 20000, 12000
_NW = 32
_NPIX = _B * _H * _W
_CHUNK = 4096

_mesh = plsc.VectorSubcoreMesh(core_axis_name="c", subcore_axis_name="s")
_params = pltpu.CompilerParams(
    needs_layout_passes=False, use_tc_tiling_on_sc=False
)


def _eclamp(x):
    return jnp.where(x < 0, jnp.minimum(x, -1e-8), jnp.maximum(x, 1e-8))


def _worker():
    wid = lax.axis_index("s") * 2 + lax.axis_index("c")
    b = wid // 8
    y0 = (wid % 8) * 64
    rb0 = b * 64 + (y0 >> 3)
    return b, y0, rb0


def _plane_off(b, rb, c):
    return ((b * 2 + c) * 64 + rb) * _CHUNK


@functools.partial(
    pl.kernel,
    mesh=_mesh,
    compiler_params=_params,
    out_type=(
        jax.ShapeDtypeStruct((_NPIX,), jnp.float32),
        jax.ShapeDtypeStruct((_NPIX * 3,), jnp.float32),
    ),
    scratch_types=[
        pltpu.VMEM((_V * 3,), jnp.float32),
        pltpu.VMEM((_F * 3,), jnp.int32),
        pltpu.VMEM((_CHUNK,), jnp.int32),
        pltpu.VMEM((_CHUNK,), jnp.int32),
        pltpu.VMEM((_CHUNK,), jnp.float32),
        pltpu.VMEM((_CHUNK,), jnp.float32),
        pltpu.VMEM((_CHUNK,), jnp.float32),
        pltpu.VMEM((_CHUNK,), jnp.float32),
        pltpu.SemaphoreType.DMA,
        pltpu.SemaphoreType.DMA,
        pltpu.SemaphoreType.DMA,
    ],
)
def _geom(v2d_hbm, vi_hbm, idx_hbm, depth_hbm, bary_hbm,
          v2d_v, vi_v, idx_a, idx_b, dep_v, q0_v, q1_v, q2_v,
          semt, semi, semo):
    b, y0, rb0 = _worker()
    loads = [
        pltpu.async_copy(v2d_hbm.at[b], v2d_v, semt),
        pltpu.async_copy(vi_hbm, vi_v, semt),
    ]
    for h in loads:
        h.wait()
    iota = lax.iota(jnp.int32, 16)

    def outs(rb):
        return [
            (dep_v, depth_hbm, rb * _CHUNK),
            (q0_v, bary_hbm, _plane_off(b, rb, 0)),
            (q1_v, bary_hbm, _plane_off(b, rb, 1)),
            (q2_v, bary_hbm, _plane_off(b, rb, 2)),
        ]

    idx_bufs = (idx_a, idx_b)
    pltpu.async_copy(idx_hbm.at[pl.ds(rb0 * _CHUNK, _CHUNK)], idx_a, semi)

    def chunk_pair(t, carry):
      for par in range(2):
        ci = t * 2 + par
        yc = y0 + ci * 8
        rb = rb0 + ci
        idx_v = idx_bufs[par]
        nxt = idx_bufs[1 - par]
        pltpu.make_async_copy(
            idx_hbm.at[pl.ds(rb * _CHUNK, _CHUNK)], idx_v, semi
        ).wait()

        @pl.when(ci < 7)
        def _():
            pltpu.async_copy(
                idx_hbm.at[pl.ds((rb + 1) * _CHUNK, _CHUNK)], nxt, semi
            )

        @pl.when(ci > 0)
        def _():
            for src, dst, off in outs(rb):
                pltpu.make_async_copy(src, dst.at[pl.ds(off, _CHUNK)], semo).wait()

        def vec_body(mi, carry2):
            for half in range(2):
                l = mi * 32 + half * 16
                f3 = idx_v[pl.ds(l, 16)] * 3
                i0 = plsc.load_gather(vi_v, [f3]) * 3
                i1 = plsc.load_gather(vi_v, [f3 + 1]) * 3
                i2 = plsc.load_gather(vi_v, [f3 + 2]) * 3
                v0x = plsc.load_gather(v2d_v, [i0])
                v0y = plsc.load_gather(v2d_v, [i0 + 1])
                v0z = plsc.load_gather(v2d_v, [i0 + 2])
                v1x = plsc.load_gather(v2d_v, [i1])
                v1y = plsc.load_gather(v2d_v, [i1 + 1])
                v1z = plsc.load_gather(v2d_v, [i1 + 2])
                v2x = plsc.load_gather(v2d_v, [i2])
                v2y = plsc.load_gather(v2d_v, [i2 + 1])
                v2z = plsc.load_gather(v2d_v, [i2 + 2])
                xb = ((l >> 10) << 7) | (l & 127)
                x = (iota + xb).astype(jnp.float32)
                y = (yc + ((l >> 7) & 7)).astype(jnp.float32)
                e1x = v1x - v0x
                e1y = v1y - v0y
                e2x = v2x - v0x
                e2y = v2y - v0y
                den = _eclamp(e1x * e2y - e1y * e2x)
                w0 = 1.0 / _eclamp(v0z)
                w1 = 1.0 / _eclamp(v1z)
                w2 = 1.0 / _eclamp(v2z)
                px = x - v0x
                py = y - v0y
                l1 = (px * e2y - py * e2x) / den
                l2 = (py * e1x - px * e1y) / den
                lam0 = 1.0 - l1 - l2
                u0 = w0 * lam0
                u1 = w1 * l1
                u2 = w2 * l2
                zi = 1.0 / _eclamp(u0 + u1 + u2)
                dep_v[pl.ds(l, 16)] = zi
                q0_v[pl.ds(l, 16)] = u0 * zi
                q1_v[pl.ds(l, 16)] = u1 * zi
                q2_v[pl.ds(l, 16)] = u2 * zi
            return carry2

        lax.fori_loop(0, _CHUNK // 32, vec_body, 0)
        for src, dst, off in outs(rb):
            pltpu.async_copy(src, dst.at[pl.ds(off, _CHUNK)], semo)
      return carry

    lax.fori_loop(0, 4, chunk_pair, 0)
    for src, dst, off in outs(rb0 + 7):
        pltpu.make_async_copy(src, dst.at[pl.ds(off, _CHUNK)], semo).wait()


@functools.partial(
    pl.kernel,
    mesh=_mesh,
    compiler_params=_params,
    out_type=jax.ShapeDtypeStruct((_NPIX * 2,), jnp.float32),
    scratch_types=[
        pltpu.VMEM((_VT * 2,), jnp.float32),
        pltpu.VMEM((_F * 3,), jnp.int32),
        pltpu.VMEM((_CHUNK,), jnp.int32),
        pltpu.VMEM((_CHUNK,), jnp.int32),
        pltpu.VMEM((_CHUNK,), jnp.float32),
        pltpu.VMEM((_CHUNK,), jnp.float32),
        pltpu.VMEM((_CHUNK,), jnp.float32),
        pltpu.VMEM((_CHUNK,), jnp.float32),
        pltpu.VMEM((_CHUNK,), jnp.float32),
        pltpu.VMEM((_CHUNK,), jnp.float32),
        pltpu.VMEM((_CHUNK * 2,), jnp.float32),
        pltpu.SemaphoreType.DMA,
        pltpu.SemaphoreType.DMA,
        pltpu.SemaphoreType.DMA,
    ],
)
def _texco(vt_hbm, vti_hbm, idx_hbm, bary_hbm, vto_hbm,
           vt_v, vti_v, idx_a, idx_b, qa0, qa1, qa2, qb0, qb1, qb2,
           out_v, semt, semi, semo):
    b, y0, rb0 = _worker()
    loads = [
        pltpu.async_copy(vt_hbm, vt_v, semt),
        pltpu.async_copy(vti_hbm, vti_v, semt),
    ]
    for h in loads:
        h.wait()

    insets = ((idx_a, qa0, qa1, qa2), (idx_b, qb0, qb1, qb2))

    def fire_ins(rb, bufs):
        pltpu.async_copy(idx_hbm.at[pl.ds(rb * _CHUNK, _CHUNK)], bufs[0], semi)
        for c in range(3):
            pltpu.async_copy(
                bary_hbm.at[pl.ds(_plane_off(b, rb, c), _CHUNK)],
                bufs[1 + c], semi)

    def wait_ins(rb, bufs):
        pltpu.make_async_copy(
            idx_hbm.at[pl.ds(rb * _CHUNK, _CHUNK)], bufs[0], semi).wait()
        for c in range(3):
            pltpu.make_async_copy(
                bary_hbm.at[pl.ds(_plane_off(b, rb, c), _CHUNK)],
                bufs[1 + c], semi).wait()

    fire_ins(rb0, insets[0])

    def chunk_pair(t, carry):
      for par in range(2):
        ci = t * 2 + par
        rb = rb0 + ci
        idx_v, q0_v, q1_v, q2_v = insets[par]
        wait_ins(rb, insets[par])

        @pl.when(ci < 7)
        def _():
            fire_ins(rb + 1, insets[1 - par])

        @pl.when(ci > 0)
        def _():
            pltpu.make_async_copy(
                out_v, vto_hbm.at[pl.ds(rb * _CHUNK * 2, _CHUNK * 2)], semo
            ).wait()

        def vec_body(mi, carry2):
            for half in range(2):
                l = mi * 32 + half * 16
                f3 = idx_v[pl.ds(l, 16)] * 3
                t0 = plsc.load_gather(vti_v, [f3]) * 2
                t1 = plsc.load_gather(vti_v, [f3 + 1]) * 2
                t2 = plsc.load_gather(vti_v, [f3 + 2]) * 2
                q0 = q0_v[pl.ds(l, 16)]
                q1 = q1_v[pl.ds(l, 16)]
                q2 = q2_v[pl.ds(l, 16)]
                vtx = (plsc.load_gather(vt_v, [t0]) * q0
                       + plsc.load_gather(vt_v, [t1]) * q1
                       + plsc.load_gather(vt_v, [t2]) * q2) * 2.0 - 1.0
                vty = (plsc.load_gather(vt_v, [t0 + 1]) * q0
                       + plsc.load_gather(vt_v, [t1 + 1]) * q1
                       + plsc.load_gather(vt_v, [t2 + 1]) * q2) * 2.0 - 1.0
                vtoff = ((l >> 7) & 7) * 1024 + (l >> 10) * 256 + (l & 127)
                out_v[pl.ds(vtoff, 16)] = vtx
                out_v[pl.ds(vtoff + 128, 16)] = vty
            return carry2

        lax.fori_loop(0, _CHUNK // 32, vec_body, 0)
        pltpu.async_copy(
            out_v, vto_hbm.at[pl.ds(rb * _CHUNK * 2, _CHUNK * 2)], semo
        )
      return carry

    lax.fori_loop(0, 4, chunk_pair, 0)
    pltpu.make_async_copy(
        out_v, vto_hbm.at[pl.ds((rb0 + 7) * _CHUNK * 2, _CHUNK * 2)], semo
    ).wait()


@functools.partial(
    pl.kernel,
    mesh=_mesh,
    compiler_params=_params,
    out_type=jax.ShapeDtypeStruct((_NPIX * 3,), jnp.float32),
    scratch_types=[
        pltpu.VMEM((_V * 3,), jnp.float32),
        pltpu.VMEM((_F * 3,), jnp.int32),
        pltpu.VMEM((_CHUNK,), jnp.int32),
        pltpu.VMEM((_CHUNK,), jnp.float32),
        pltpu.VMEM((_CHUNK,), jnp.float32),
        pltpu.VMEM((_CHUNK,), jnp.float32),
        pltpu.VMEM((_CHUNK,), jnp.float32),
        pltpu.VMEM((_CHUNK,), jnp.float32),
        pltpu.VMEM((_CHUNK,), jnp.float32),
        pltpu.SemaphoreType.DMA,
        pltpu.SemaphoreType.DMA,
        pltpu.SemaphoreType.DMA,
    ],
)
def _normals(vn_hbm, vi_hbm, idx_hbm, bary_hbm, vno_hbm,
             vn_v, vi_v, idx_v, q0_v, q1_v, q2_v, n0_v, n1_v, n2_v,
             semt, semi, semo):
    b, y0, rb0 = _worker()
    loads = [
        pltpu.async_copy(vn_hbm.at[b], vn_v, semt),
        pltpu.async_copy(vi_hbm, vi_v, semt),
    ]
    for h in loads:
        h.wait()

    def outs(rb):
        return [
            (n0_v, _plane_off(b, rb, 0)),
            (n1_v, _plane_off(b, rb, 1)),
            (n2_v, _plane_off(b, rb, 2)),
        ]

    def chunk_body(ci, carry):
        rb = rb0 + ci
        ins = [
            pltpu.async_copy(idx_hbm.at[pl.ds(rb * _CHUNK, _CHUNK)], idx_v, semi),
            pltpu.async_copy(
                bary_hbm.at[pl.ds(_plane_off(b, rb, 0), _CHUNK)], q0_v, semi),
            pltpu.async_copy(
                bary_hbm.at[pl.ds(_plane_off(b, rb, 1), _CHUNK)], q1_v, semi),
            pltpu.async_copy(
                bary_hbm.at[pl.ds(_plane_off(b, rb, 2), _CHUNK)], q2_v, semi),
        ]
        for h in ins:
            h.wait()

        @pl.when(ci > 0)
        def _():
            for src, off in outs(rb):
                pltpu.make_async_copy(
                    src, vno_hbm.at[pl.ds(off, _CHUNK)], semo
                ).wait()

        def vec_body(mi, carry2):
            for half in range(2):
                l = mi * 32 + half * 16
                f3 = idx_v[pl.ds(l, 16)] * 3
                i0 = plsc.load_gather(vi_v, [f3]) * 3
                i1 = plsc.load_gather(vi_v, [f3 + 1]) * 3
                i2 = plsc.load_gather(vi_v, [f3 + 2]) * 3
                q0 = q0_v[pl.ds(l, 16)]
                q1 = q1_v[pl.ds(l, 16)]
                q2 = q2_v[pl.ds(l, 16)]
                n0_v[pl.ds(l, 16)] = (
                    plsc.load_gather(vn_v, [i0]) * q0
                    + plsc.load_gather(vn_v, [i1]) * q1
                    + plsc.load_gather(vn_v, [i2]) * q2)
                n1_v[pl.ds(l, 16)] = (
                    plsc.load_gather(vn_v, [i0 + 1]) * q0
                    + plsc.load_gather(vn_v, [i1 + 1]) * q1
                    + plsc.load_gather(vn_v, [i2 + 1]) * q2)
                n2_v[pl.ds(l, 16)] = (
                    plsc.load_gather(vn_v, [i0 + 2]) * q0
                    + plsc.load_gather(vn_v, [i1 + 2]) * q1
                    + plsc.load_gather(vn_v, [i2 + 2]) * q2)
            return carry2

        lax.fori_loop(0, _CHUNK // 32, vec_body, 0)
        for src, off in outs(rb):
            pltpu.async_copy(src, vno_hbm.at[pl.ds(off, _CHUNK)], semo)
        return carry

    lax.fori_loop(0, 8, chunk_body, 0)
    for src, off in outs(rb0 + 7):
        pltpu.make_async_copy(src, vno_hbm.at[pl.ds(off, _CHUNK)], semo).wait()


def kernel(v2d, vt, vn, vi, vti, index_img):
    vi_f = vi.reshape(-1)
    vti_f = vti.reshape(-1)
    v2d_f = v2d.reshape(_B, _V * 3)
    vn_f = vn.reshape(_B, _V * 3)
    vt_f = vt.reshape(-1)
    idx_tiled = index_img.reshape(_B, 64, 8, 4, 128)
    idx_tiled = idx_tiled.transpose(0, 1, 3, 2, 4).reshape(-1)
    depth, bary = _geom(v2d_f, vi_f, idx_tiled)
    vto = _texco(vt_f, vti_f, idx_tiled, bary)
    vno = _normals(vn_f, vi_f, idx_tiled, bary)
    depth = depth.reshape(_B, 64, 4, 8, 128).transpose(0, 1, 3, 2, 4)
    depth = depth.reshape(_B, _H, _W)
    bary = bary.reshape(_B, 3, 64, 4, 8, 128).transpose(0, 2, 4, 3, 5, 1)
    bary = bary.reshape(_B, _H, _W, 3)
    vno = vno.reshape(_B, 3, 64, 4, 8, 128).transpose(0, 2, 4, 3, 5, 1)
    vno = vno.reshape(_B, _H, _W, 3)
    vto = vto.reshape(_B, _H, 4, 2, 128).transpose(0, 1, 2, 4, 3)
    vto = vto.reshape(_B, _H, _W, 2)
    return depth, bary, vto, vno
